# baseline (speedup 1.0000x reference)
import functools
import jax
import jax.numpy as jnp
from jax import lax
from jax.experimental import pallas as pl
from jax.experimental.pallas import tpu as pltpu
from jax.experimental.pallas import tpu_sc as plsc

NUM_GRAPHS = 64
_EPS = 1e-5
_HI = lax.Precision.HIGHEST

_NC = 2
_NS = 16
_L = 16
_NW = _NC * _NS


def _dyn_gather(v, idx):
    return lax.gather(
        v, idx[:, None],
        lax.GatherDimensionNumbers(offset_dims=(), collapsed_slice_dims=(0,),
                                   start_index_map=(0,)),
        (1,), mode=lax.GatherScatterMode.PROMISE_IN_BOUNDS)


def _sc_agg(h, src_p, dst_p, npt, chunk, sub):
    n, d = h.shape
    n_pad = _NW * npt
    e_pad = src_p.shape[0]
    nchunks = e_pad // chunk
    mesh = plsc.VectorSubcoreMesh(core_axis_name="c", subcore_axis_name="s")

    @functools.partial(
        pl.kernel, mesh=mesh,
        compiler_params=pltpu.CompilerParams(needs_layout_passes=False),
        out_type=jax.ShapeDtypeStruct((n_pad * d,), jnp.float32),
        scratch_types=[
            pltpu.VMEM((npt * d + d,), jnp.float32),
            pltpu.VMEM((chunk,), jnp.int32),
            pltpu.VMEM((chunk,), jnp.int32),
            pltpu.VMEM((chunk + sub,), jnp.int32),
            pltpu.VMEM((chunk + sub,), jnp.int32),
            pltpu.VMEM((sub,), jnp.int32),
            pltpu.VMEM((sub, d), jnp.float32),
            pltpu.SemaphoreType.DMA,
        ],
    )
    def agg_kernel(h_hbm, src_hbm, dst_hbm, out_hbm,
                   acc, dstb, srcb, sel_slot, sel_src, idxb, rows, sem):
        wid = lax.axis_index("s") * _NC + lax.axis_index("c")
        lo = wid * npt
        neg = jnp.full((_L,), -jnp.inf, jnp.float32)
        zeros16 = jnp.zeros((_L,), jnp.int32)
        lane_iota = lax.iota(jnp.int32, _L)

        def init_body(i, _):
            acc[pl.ds(i * _L, _L)] = neg
            return 0
        lax.fori_loop(0, (npt * d + d) // _L, init_body, 0, unroll=4)

        def chunk_body(ci, _):
            base_e = ci * chunk
            pltpu.sync_copy(dst_hbm.at[pl.ds(base_e, chunk)], dstb)
            pltpu.sync_copy(src_hbm.at[pl.ds(base_e, chunk)], srcb)

            def scan_body(g, cnt):
                d16 = dstb[pl.ds(g * _L, _L)]
                s16 = srcb[pl.ds(g * _L, _L)]
                t = d16 - lo
                m = (d16 >= lo) & (t < npt)
                plsc.store_compressed(sel_slot.at[pl.ds(cnt, _L)], t, mask=m)
                plsc.store_compressed(sel_src.at[pl.ds(cnt, _L)], s16, mask=m)
                pc = plsc.all_reduce_population_count(m)
                return cnt + jnp.max(pc)

            cnt = lax.fori_loop(0, chunk // _L, scan_body, jnp.int32(0),
                                unroll=2)

            def clr_body(k, _):
                sel_src[pl.ds(cnt + k * _L, _L)] = zeros16
                return 0
            lax.fori_loop(0, sub // _L, clr_body, 0)

            nsub = (cnt + (sub - 1)) // sub

            def sub_body(b, _):
                sb = b * sub

                def cpidx_body(k, _):
                    idxb[pl.ds(k * _L, _L)] = sel_src[pl.ds(sb + k * _L, _L)]
                    return 0
                lax.fori_loop(0, sub // _L, cpidx_body, 0)
                pltpu.async_copy(h_hbm.at[idxb], rows, sem).wait()

                rem = jnp.minimum(cnt - sb, sub)

                def edge_body(jj, _):
                    q = (jj // _L) * _L
                    lane = jj - q
                    slots16 = sel_slot[pl.ds(sb + q, _L)]
                    sl = _dyn_gather(slots16, zeros16 + lane)
                    addr = sl * d + lane_iota
                    for k in range(d // _L):
                        a_idx = addr + (k * _L)
                        cur = plsc.load_gather(acc, [a_idx])
                        hv = rows[jj, pl.ds(k * _L, _L)]
                        plsc.store_scatter(acc, [a_idx],
                                           jnp.maximum(cur, hv))
                    return 0
                lax.fori_loop(0, rem, edge_body, 0)
                return 0

            lax.fori_loop(0, nsub, sub_body, 0)
            return 0

        lax.fori_loop(0, nchunks, chunk_body, 0)
        pltpu.sync_copy(acc.at[pl.ds(0, npt * d)],
                        out_hbm.at[pl.ds(lo * d, npt * d)])

    return agg_kernel(h, src_p, dst_p)


def _seg(Mt, v):
    return lax.dot_general(Mt, v, (((1,), (0,)), ((), ())), precision=_HI)


def _bcast(Mt, v):
    return lax.dot_general(Mt, v, (((0,), (0,)), ((), ())), precision=_HI)


def _graph_norm(x, Mt, inv_cnt, w, b, ms):
    mean = _seg(Mt, x) * inv_cnt
    t = x - _bcast(Mt, mean) * ms
    var = _seg(Mt, t * t) * inv_cnt
    std = jnp.sqrt(var + _EPS)
    return w * t / _bcast(Mt, std) + b


def _one_hot_t(batch1n, g):
    gi = lax.broadcasted_iota(jnp.int32, (g, 1), 0)
    return (batch1n == gi).astype(jnp.float32)


def _inv_cnt(Mt, n):
    ones = jnp.ones((n, 1), jnp.float32)
    cnt = lax.dot_general(Mt, ones, (((1,), (0,)), ((), ())), precision=_HI)
    return 1.0 / jnp.maximum(cnt, 1.0)


def _k1_body(x_ref, batch_ref, w_ref, b_ref, ms_ref, Wr_ref, bl_ref,
             h_ref, r_ref):
    x = x_ref[...]
    Mt = _one_hot_t(batch_ref[...], NUM_GRAPHS)
    h = _graph_norm(x, Mt, _inv_cnt(Mt, x.shape[0]),
                    w_ref[...], b_ref[...], ms_ref[...])
    h_ref[...] = h
    r_ref[...] = x + jnp.dot(h, Wr_ref[...],
                             preferred_element_type=jnp.float32) + bl_ref[...]


def _k3_body(agg_ref, r_ref, Wl_ref, batch_ref, w_ref, b_ref, ms_ref,
             out_ref):
    agg = agg_ref[...]
    agg = jnp.where(jnp.isfinite(agg), agg, 0.0)
    pre = jnp.dot(agg, Wl_ref[...], preferred_element_type=jnp.float32) \
        + r_ref[...]
    h2 = jnp.maximum(pre, 0.0)
    Mt = _one_hot_t(batch_ref[...], NUM_GRAPHS)
    out_ref[...] = _graph_norm(h2, Mt, _inv_cnt(Mt, h2.shape[0]),
                               w_ref[...], b_ref[...], ms_ref[...])


def kernel(x, edge_index, batch, W_l, b_l, W_r,
           gn1_w, gn1_b, gn1_ms, gn2_w, gn2_b, gn2_ms):
    n, d = x.shape
    e = edge_index.shape[1]
    batch1n = batch.reshape(1, n)
    row = lambda v: v.reshape(1, d)

    h, r = pl.pallas_call(
        _k1_body,
        out_shape=(jax.ShapeDtypeStruct((n, d), jnp.float32),
                   jax.ShapeDtypeStruct((n, d), jnp.float32)),
    )(x, batch1n, row(gn1_w), row(gn1_b), row(gn1_ms), W_r, row(b_l))

    npt = (n + _NW - 1) // _NW
    npt = ((npt + 7) // 8) * 8
    chunk = 4096
    e_pad = ((e + chunk - 1) // chunk) * chunk
    src_p = jnp.concatenate(
        [edge_index[0], jnp.zeros((e_pad - e,), jnp.int32)])
    dst_p = jnp.concatenate(
        [edge_index[1], jnp.full((e_pad - e,), 1 << 20, jnp.int32)])
    agg_flat = _sc_agg(h, src_p, dst_p, npt, chunk, 64)
    agg = agg_flat.reshape(_NW * npt, d)[:n]

    out = pl.pallas_call(
        _k3_body,
        out_shape=jax.ShapeDtypeStruct((n, d), jnp.float32),
    )(agg, r, W_l, batch1n, row(gn2_w), row(gn2_b), row(gn2_ms))
    return out


# speedup vs baseline: 6.0985x; 6.0985x over previous
import functools
import jax
import jax.numpy as jnp
from jax import lax
from jax.experimental import pallas as pl
from jax.experimental.pallas import tpu as pltpu
from jax.experimental.pallas import tpu_sc as plsc

NUM_GRAPHS = 64
_EPS = 1e-5
_HI = lax.Precision.HIGHEST

_NC = 2
_NS = 16
_L = 16
_NW = _NC * _NS


def _dyn_gather(v, idx):
    return lax.gather(
        v, idx[:, None],
        lax.GatherDimensionNumbers(offset_dims=(), collapsed_slice_dims=(0,),
                                   start_index_map=(0,)),
        (1,), mode=lax.GatherScatterMode.PROMISE_IN_BOUNDS)


def _sc_agg(h, src_p, dst_p, npt, chunk, sub):
    n, d = h.shape
    n_pad = _NW * npt
    e_pad = src_p.shape[0]
    nchunks = e_pad // chunk
    ng = chunk // _L
    nk = d // _L
    mesh = plsc.VectorSubcoreMesh(core_axis_name="c", subcore_axis_name="s")

    @functools.partial(
        pl.kernel, mesh=mesh,
        compiler_params=pltpu.CompilerParams(needs_layout_passes=False),
        out_type=jax.ShapeDtypeStruct((n_pad * d,), jnp.float32),
        scratch_types=[
            pltpu.VMEM((npt * d + d,), jnp.float32),
            pltpu.VMEM((2, chunk), jnp.int32),
            pltpu.VMEM((2, chunk), jnp.int32),
            pltpu.VMEM((chunk + 2 * sub,), jnp.int32),
            pltpu.VMEM((chunk + 2 * sub,), jnp.int32),
            pltpu.VMEM((2, sub), jnp.int32),
            pltpu.VMEM((2, sub, d), jnp.float32),
            pltpu.SemaphoreType.DMA,
            pltpu.SemaphoreType.DMA,
        ],
    )
    def agg_kernel(h_hbm, src_hbm, dst_hbm, out_hbm,
                   acc, dstb, srcb, pslot, psrc, idxb, rows, csem, gsem):
        wid = lax.axis_index("s") * _NC + lax.axis_index("c")
        lo = wid * npt
        neg = jnp.full((_L,), -jnp.inf, jnp.float32)
        zeros16 = jnp.zeros((_L,), jnp.int32)
        lane_iota = lax.iota(jnp.int32, _L)

        def init_body(i, _):
            acc[pl.ds(i * _L, _L)] = neg
            return 0
        lax.fori_loop(0, (npt * d + d) // _L, init_body, 0, unroll=8)
        for k2 in range(2 * sub // _L):
            psrc[pl.ds(k2 * _L, _L)] = zeros16

        def start_chunk(ci, par):
            pltpu.async_copy(dst_hbm.at[pl.ds(ci * chunk, chunk)],
                             dstb.at[par], csem)
            pltpu.async_copy(src_hbm.at[pl.ds(ci * chunk, chunk)],
                             srcb.at[par], csem)

        def wait_chunk(par):
            pltpu.make_async_copy(dst_hbm.at[pl.ds(0, chunk)],
                                  dstb.at[par], csem).wait()
            pltpu.make_async_copy(src_hbm.at[pl.ds(0, chunk)],
                                  srcb.at[par], csem).wait()

        def start_gather(base, par):
            def cp(k, _):
                v = psrc[pl.ds(base + k * _L, _L)]
                idxb[par, pl.ds(k * _L, _L)] = jnp.clip(v, 0, n - 1)
                return 0
            lax.fori_loop(0, sub // _L, cp, 0, unroll=sub // _L)
            pltpu.async_copy(h_hbm.at[idxb.at[par]], rows.at[par], gsem)

        def wait_gather(par):
            pltpu.make_async_copy(h_hbm.at[idxb.at[par]],
                                  rows.at[par], gsem).wait()

        def accum(base, par, jj):
            q16 = (jj // _L) * _L
            lane = jj - q16
            slots16 = pslot[pl.ds(base + q16, _L)]
            sl = _dyn_gather(slots16, zeros16 + lane)
            addr = sl * d + lane_iota
            idxs = [addr + (k * _L) for k in range(nk)]
            curs = [plsc.load_gather(acc, [ix]) for ix in idxs]
            hvs = [rows[par, jj, pl.ds(k * _L, _L)] for k in range(nk)]
            for k in range(nk):
                plsc.store_scatter(acc, [idxs[k]],
                                   jnp.maximum(curs[k], hvs[k]))

        start_chunk(0, 0)

        def chunk_body(ci, cnt):
            par = lax.rem(ci, 2)
            wait_chunk(par)

            @pl.when(ci + 1 < nchunks)
            def _pref():
                start_chunk(ci + 1, lax.rem(ci + 1, 2))

            def scan_body(g, cnt):
                d16 = dstb[par, pl.ds(g * _L, _L)]
                s16 = srcb[par, pl.ds(g * _L, _L)]
                t = d16 - lo
                m = (d16 >= lo) & (t < npt)
                plsc.store_compressed(pslot.at[pl.ds(cnt, _L)], t, mask=m)
                plsc.store_compressed(psrc.at[pl.ds(cnt, _L)], s16, mask=m)
                pc = plsc.all_reduce_population_count(m)
                return cnt + pc[0]

            cnt = lax.fori_loop(0, ng, scan_body, cnt, unroll=4)
            nsub = cnt // sub

            @pl.when(nsub > 0)
            def _drain():
                start_gather(0, 0)

                def sub_body(b, _):
                    gpar = lax.rem(b, 2)
                    wait_gather(gpar)

                    @pl.when(b + 1 < nsub)
                    def _p():
                        start_gather((b + 1) * sub, lax.rem(b + 1, 2))

                    def edge_body(jj, _):
                        accum(b * sub, gpar, jj)
                        return 0
                    lax.fori_loop(0, sub, edge_body, 0, unroll=4)
                    return 0

                lax.fori_loop(0, nsub, sub_body, 0)

            done = nsub * sub
            rem = cnt - done

            @pl.when(done > 0)
            def _mv():
                def mv(k, _):
                    v1 = pslot[pl.ds(done + k * _L, _L)]
                    v2 = psrc[pl.ds(done + k * _L, _L)]
                    pslot[pl.ds(k * _L, _L)] = v1
                    psrc[pl.ds(k * _L, _L)] = v2
                    return 0
                lax.fori_loop(0, sub // _L, mv, 0, unroll=sub // _L)
            return rem

        cnt = lax.fori_loop(0, nchunks, chunk_body, jnp.int32(0))

        @pl.when(cnt > 0)
        def _flush():
            start_gather(0, 0)
            wait_gather(0)

            def edge_body(jj, _):
                accum(0, 0, jj)
                return 0
            lax.fori_loop(0, cnt, edge_body, 0)

        pltpu.sync_copy(acc.at[pl.ds(0, npt * d)],
                        out_hbm.at[pl.ds(lo * d, npt * d)])

    return agg_kernel(h, src_p, dst_p)


def _seg(Mt, v):
    return lax.dot_general(Mt, v, (((1,), (0,)), ((), ())), precision=_HI)


def _bcast(Mt, v):
    return lax.dot_general(Mt, v, (((0,), (0,)), ((), ())), precision=_HI)


def _graph_norm(x, Mt, inv_cnt, w, b, ms):
    mean = _seg(Mt, x) * inv_cnt
    t = x - _bcast(Mt, mean) * ms
    var = _seg(Mt, t * t) * inv_cnt
    std = jnp.sqrt(var + _EPS)
    return w * t / _bcast(Mt, std) + b


def _one_hot_t(batch1n, g):
    gi = lax.broadcasted_iota(jnp.int32, (g, 1), 0)
    return (batch1n == gi).astype(jnp.float32)


def _inv_cnt(Mt, n):
    ones = jnp.ones((n, 1), jnp.float32)
    cnt = lax.dot_general(Mt, ones, (((1,), (0,)), ((), ())), precision=_HI)
    return 1.0 / jnp.maximum(cnt, 1.0)


def _k1_body(x_ref, batch_ref, w_ref, b_ref, ms_ref, Wr_ref, bl_ref,
             h_ref, r_ref):
    x = x_ref[...]
    Mt = _one_hot_t(batch_ref[...], NUM_GRAPHS)
    h = _graph_norm(x, Mt, _inv_cnt(Mt, x.shape[0]),
                    w_ref[...], b_ref[...], ms_ref[...])
    h_ref[...] = h
    r_ref[...] = x + jnp.dot(h, Wr_ref[...],
                             preferred_element_type=jnp.float32) + bl_ref[...]


def _k3_body(agg_ref, r_ref, Wl_ref, batch_ref, w_ref, b_ref, ms_ref,
             out_ref):
    agg = agg_ref[...]
    agg = jnp.where(jnp.isfinite(agg), agg, 0.0)
    pre = jnp.dot(agg, Wl_ref[...], preferred_element_type=jnp.float32) \
        + r_ref[...]
    h2 = jnp.maximum(pre, 0.0)
    Mt = _one_hot_t(batch_ref[...], NUM_GRAPHS)
    out_ref[...] = _graph_norm(h2, Mt, _inv_cnt(Mt, h2.shape[0]),
                               w_ref[...], b_ref[...], ms_ref[...])


def kernel(x, edge_index, batch, W_l, b_l, W_r,
           gn1_w, gn1_b, gn1_ms, gn2_w, gn2_b, gn2_ms):
    n, d = x.shape
    e = edge_index.shape[1]
    batch1n = batch.reshape(1, n)
    row = lambda v: v.reshape(1, d)

    h, r = pl.pallas_call(
        _k1_body,
        out_shape=(jax.ShapeDtypeStruct((n, d), jnp.float32),
                   jax.ShapeDtypeStruct((n, d), jnp.float32)),
    )(x, batch1n, row(gn1_w), row(gn1_b), row(gn1_ms), W_r, row(b_l))

    npt = (n + _NW - 1) // _NW
    npt = ((npt + 7) // 8) * 8
    chunk = 4096
    e_pad = ((e + chunk - 1) // chunk) * chunk
    src_p = jnp.concatenate(
        [edge_index[0], jnp.zeros((e_pad - e,), jnp.int32)])
    dst_p = jnp.concatenate(
        [edge_index[1], jnp.full((e_pad - e,), 1 << 20, jnp.int32)])
    agg_flat = _sc_agg(h, src_p, dst_p, npt, chunk, 64)
    agg = agg_flat.reshape(_NW * npt, d)[:n]

    out = pl.pallas_call(
        _k3_body,
        out_shape=jax.ShapeDtypeStruct((n, d), jnp.float32),
    )(agg, r, W_l, batch1n, row(gn2_w), row(gn2_b), row(gn2_ms))
    return out


# speedup vs baseline: 6.6587x; 1.0919x over previous
import functools
import jax
import jax.numpy as jnp
from jax import lax
from jax.experimental import pallas as pl
from jax.experimental.pallas import tpu as pltpu
from jax.experimental.pallas import tpu_sc as plsc

NUM_GRAPHS = 64
_EPS = 1e-5
_HI = lax.Precision.HIGHEST

_NC = 2
_NS = 16
_L = 16
_NW = _NC * _NS


def _dyn_gather(v, idx):
    return lax.gather(
        v, idx[:, None],
        lax.GatherDimensionNumbers(offset_dims=(), collapsed_slice_dims=(0,),
                                   start_index_map=(0,)),
        (1,), mode=lax.GatherScatterMode.PROMISE_IN_BOUNDS)


def _sc_agg(h, src_p, dst_p, npt, chunk, sub):
    n, d = h.shape
    n_pad = _NW * npt
    e_pad = src_p.shape[0]
    nchunks = e_pad // chunk
    ng = chunk // _L
    nk = d // _L
    mesh = plsc.VectorSubcoreMesh(core_axis_name="c", subcore_axis_name="s")

    @functools.partial(
        pl.kernel, mesh=mesh,
        compiler_params=pltpu.CompilerParams(needs_layout_passes=False),
        out_type=jax.ShapeDtypeStruct((n_pad * d,), jnp.float32),
        scratch_types=[
            pltpu.VMEM((npt * d + d,), jnp.float32),
            pltpu.VMEM((2, chunk), jnp.int32),
            pltpu.VMEM((2, chunk), jnp.int32),
            pltpu.VMEM((chunk + 2 * sub,), jnp.int32),
            pltpu.VMEM((chunk + 2 * sub,), jnp.int32),
            pltpu.VMEM((2, sub), jnp.int32),
            pltpu.VMEM((2, sub, d), jnp.float32),
            pltpu.SemaphoreType.DMA,
            pltpu.SemaphoreType.DMA,
        ],
    )
    def agg_kernel(h_hbm, src_hbm, dst_hbm, out_hbm,
                   acc, dstb, srcb, pslot, psrc, idxb, rows, csem, gsem):
        wid = lax.axis_index("s") * _NC + lax.axis_index("c")
        lo = wid * npt
        neg = jnp.full((_L,), -jnp.inf, jnp.float32)
        zeros16 = jnp.zeros((_L,), jnp.int32)
        lane_iota = lax.iota(jnp.int32, _L)

        def init_body(i, _):
            acc[pl.ds(i * _L, _L)] = neg
            return 0
        lax.fori_loop(0, (npt * d + d) // _L, init_body, 0, unroll=8)
        for k2 in range(2 * sub // _L):
            psrc[pl.ds(k2 * _L, _L)] = zeros16

        def start_chunk(ci, par):
            pltpu.async_copy(dst_hbm.at[pl.ds(ci * chunk, chunk)],
                             dstb.at[par], csem)
            pltpu.async_copy(src_hbm.at[pl.ds(ci * chunk, chunk)],
                             srcb.at[par], csem)

        def wait_chunk(par):
            pltpu.make_async_copy(dst_hbm.at[pl.ds(0, chunk)],
                                  dstb.at[par], csem).wait()
            pltpu.make_async_copy(src_hbm.at[pl.ds(0, chunk)],
                                  srcb.at[par], csem).wait()

        def start_gather(base, par):
            def cp(k, _):
                v = psrc[pl.ds(base + k * _L, _L)]
                idxb[par, pl.ds(k * _L, _L)] = jnp.clip(v, 0, n - 1)
                return 0
            lax.fori_loop(0, sub // _L, cp, 0, unroll=sub // _L)
            pltpu.async_copy(h_hbm.at[idxb.at[par]], rows.at[par], gsem)

        def wait_gather(par):
            pltpu.make_async_copy(h_hbm.at[idxb.at[par]],
                                  rows.at[par], gsem).wait()

        def accum_edge(slots16, par, jj, lane):
            sl = _dyn_gather(slots16, zeros16 + lane)
            addr = sl * d + lane_iota
            idxs = [addr + (k * _L) for k in range(nk)]
            curs = [plsc.load_gather(acc, [ix]) for ix in idxs]
            hvs = [rows[par, jj, pl.ds(k * _L, _L)] for k in range(nk)]
            for k in range(nk):
                plsc.store_scatter(acc, [idxs[k]],
                                   jnp.maximum(curs[k], hvs[k]))

        def accum_group(base, par, gq):
            slots16 = pslot[pl.ds(base + gq * _L, _L)]
            for lane in range(_L):
                accum_edge(slots16, par, gq * _L + lane, lane)

        def accum(base, par, jj):
            q16 = (jj // _L) * _L
            lane = jj - q16
            slots16 = pslot[pl.ds(base + q16, _L)]
            accum_edge(slots16, par, jj, lane)

        start_chunk(0, 0)

        def chunk_body(ci, cnt):
            par = lax.rem(ci, 2)
            wait_chunk(par)

            @pl.when(ci + 1 < nchunks)
            def _pref():
                start_chunk(ci + 1, lax.rem(ci + 1, 2))

            def scan_body(g, cnt):
                d16 = dstb[par, pl.ds(g * _L, _L)]
                s16 = srcb[par, pl.ds(g * _L, _L)]
                t = d16 - lo
                m = (d16 >= lo) & (t < npt)
                plsc.store_compressed(pslot.at[pl.ds(cnt, _L)], t, mask=m)
                plsc.store_compressed(psrc.at[pl.ds(cnt, _L)], s16, mask=m)
                pc = plsc.all_reduce_population_count(m)
                return cnt + pc[0]

            cnt = lax.fori_loop(0, ng, scan_body, cnt, unroll=8)
            nsub = cnt // sub

            @pl.when(nsub > 0)
            def _drain():
                start_gather(0, 0)

                def sub_body(b, _):
                    gpar = lax.rem(b, 2)
                    wait_gather(gpar)

                    @pl.when(b + 1 < nsub)
                    def _p():
                        start_gather((b + 1) * sub, lax.rem(b + 1, 2))

                    def grp_body(gq, _):
                        accum_group(b * sub, gpar, gq)
                        return 0
                    lax.fori_loop(0, sub // _L, grp_body, 0)
                    return 0

                lax.fori_loop(0, nsub, sub_body, 0)

            done = nsub * sub
            rem = cnt - done

            @pl.when(done > 0)
            def _mv():
                def mv(k, _):
                    v1 = pslot[pl.ds(done + k * _L, _L)]
                    v2 = psrc[pl.ds(done + k * _L, _L)]
                    pslot[pl.ds(k * _L, _L)] = v1
                    psrc[pl.ds(k * _L, _L)] = v2
                    return 0
                lax.fori_loop(0, sub // _L, mv, 0, unroll=sub // _L)
            return rem

        cnt = lax.fori_loop(0, nchunks, chunk_body, jnp.int32(0))

        @pl.when(cnt > 0)
        def _flush():
            start_gather(0, 0)
            wait_gather(0)

            def edge_body(jj, _):
                accum(0, 0, jj)
                return 0
            lax.fori_loop(0, cnt, edge_body, 0)

        pltpu.sync_copy(acc.at[pl.ds(0, npt * d)],
                        out_hbm.at[pl.ds(lo * d, npt * d)])

    return agg_kernel(h, src_p, dst_p)


def _seg(Mt, v):
    return lax.dot_general(Mt, v, (((1,), (0,)), ((), ())), precision=_HI)


def _bcast(Mt, v):
    return lax.dot_general(Mt, v, (((0,), (0,)), ((), ())), precision=_HI)


def _graph_norm(x, Mt, inv_cnt, w, b, ms):
    mean = _seg(Mt, x) * inv_cnt
    t = x - _bcast(Mt, mean) * ms
    var = _seg(Mt, t * t) * inv_cnt
    std = jnp.sqrt(var + _EPS)
    return w * t / _bcast(Mt, std) + b


def _one_hot_t(batch1n, g):
    gi = lax.broadcasted_iota(jnp.int32, (g, 1), 0)
    return (batch1n == gi).astype(jnp.float32)


def _inv_cnt(Mt, n):
    ones = jnp.ones((n, 1), jnp.float32)
    cnt = lax.dot_general(Mt, ones, (((1,), (0,)), ((), ())), precision=_HI)
    return 1.0 / jnp.maximum(cnt, 1.0)


def _k1_body(x_ref, batch_ref, w_ref, b_ref, ms_ref, Wr_ref, bl_ref,
             h_ref, r_ref):
    x = x_ref[...]
    Mt = _one_hot_t(batch_ref[...], NUM_GRAPHS)
    h = _graph_norm(x, Mt, _inv_cnt(Mt, x.shape[0]),
                    w_ref[...], b_ref[...], ms_ref[...])
    h_ref[...] = h
    r_ref[...] = x + jnp.dot(h, Wr_ref[...],
                             preferred_element_type=jnp.float32) + bl_ref[...]


def _k3_body(agg_ref, r_ref, Wl_ref, batch_ref, w_ref, b_ref, ms_ref,
             out_ref):
    agg = agg_ref[...]
    agg = jnp.where(jnp.isfinite(agg), agg, 0.0)
    pre = jnp.dot(agg, Wl_ref[...], preferred_element_type=jnp.float32) \
        + r_ref[...]
    h2 = jnp.maximum(pre, 0.0)
    Mt = _one_hot_t(batch_ref[...], NUM_GRAPHS)
    out_ref[...] = _graph_norm(h2, Mt, _inv_cnt(Mt, h2.shape[0]),
                               w_ref[...], b_ref[...], ms_ref[...])


def kernel(x, edge_index, batch, W_l, b_l, W_r,
           gn1_w, gn1_b, gn1_ms, gn2_w, gn2_b, gn2_ms):
    n, d = x.shape
    e = edge_index.shape[1]
    batch1n = batch.reshape(1, n)
    row = lambda v: v.reshape(1, d)

    h, r = pl.pallas_call(
        _k1_body,
        out_shape=(jax.ShapeDtypeStruct((n, d), jnp.float32),
                   jax.ShapeDtypeStruct((n, d), jnp.float32)),
    )(x, batch1n, row(gn1_w), row(gn1_b), row(gn1_ms), W_r, row(b_l))

    npt = (n + _NW - 1) // _NW
    npt = ((npt + 7) // 8) * 8
    chunk = 8192
    e_pad = ((e + chunk - 1) // chunk) * chunk
    src_p = jnp.concatenate(
        [edge_index[0], jnp.zeros((e_pad - e,), jnp.int32)])
    dst_p = jnp.concatenate(
        [edge_index[1], jnp.full((e_pad - e,), 1 << 20, jnp.int32)])
    agg_flat = _sc_agg(h, src_p, dst_p, npt, chunk, 64)
    agg = agg_flat.reshape(_NW * npt, d)[:n]

    out = pl.pallas_call(
        _k3_body,
        out_shape=jax.ShapeDtypeStruct((n, d), jnp.float32),
    )(agg, r, W_l, batch1n, row(gn2_w), row(gn2_b), row(gn2_ms))
    return out


# speedup vs baseline: 8.7458x; 1.3134x over previous
import functools
import jax
import jax.numpy as jnp
from jax import lax
from jax.experimental import pallas as pl
from jax.experimental.pallas import tpu as pltpu
from jax.experimental.pallas import tpu_sc as plsc

NUM_GRAPHS = 64
_EPS = 1e-5
_HI = lax.Precision.HIGHEST

_NC = 2
_NS = 16
_L = 16
_NW = _NC * _NS


def _dyn_gather(v, idx):
    return lax.gather(
        v, idx[:, None],
        lax.GatherDimensionNumbers(offset_dims=(), collapsed_slice_dims=(0,),
                                   start_index_map=(0,)),
        (1,), mode=lax.GatherScatterMode.PROMISE_IN_BOUNDS)


def _sc_agg(h, src_p, dst_p, npt, chunk, sub):
    n, d = h.shape
    n_pad = _NW * npt
    e_pad = src_p.shape[0]
    nchunks = e_pad // chunk
    ng = chunk // _L
    nk = d // _L
    mesh = plsc.VectorSubcoreMesh(core_axis_name="c", subcore_axis_name="s")

    @functools.partial(
        pl.kernel, mesh=mesh,
        compiler_params=pltpu.CompilerParams(needs_layout_passes=False),
        out_type=jax.ShapeDtypeStruct((n_pad * d,), jnp.float32),
        scratch_types=[
            pltpu.VMEM((npt * d + d,), jnp.float32),
            pltpu.VMEM((2, chunk), jnp.int32),
            pltpu.VMEM((2, chunk), jnp.int32),
            pltpu.VMEM((chunk + 2 * sub,), jnp.int32),
            pltpu.VMEM((2, sub), jnp.int32),
            pltpu.VMEM((2, sub, d), jnp.float32),
            pltpu.SemaphoreType.DMA,
            pltpu.SemaphoreType.DMA,
        ],
    )
    def agg_kernel(h_hbm, src_hbm, dst_hbm, out_hbm,
                   acc, dstb, srcb, pend, idxb, rows, csem, gsem):
        wid = lax.axis_index("s") * _NC + lax.axis_index("c")
        lo = wid * npt
        neg = jnp.full((_L,), -jnp.inf, jnp.float32)
        zeros16 = jnp.zeros((_L,), jnp.int32)
        lane_iota = lax.iota(jnp.int32, _L)

        def init_body(i, _):
            acc[pl.ds(i * _L, _L)] = neg
            return 0
        lax.fori_loop(0, (npt * d + d) // _L, init_body, 0, unroll=8)
        for k2 in range(2 * sub // _L):
            pend[pl.ds(k2 * _L, _L)] = zeros16

        def start_chunk(ci, par):
            pltpu.async_copy(dst_hbm.at[pl.ds(ci * chunk, chunk)],
                             dstb.at[par], csem)
            pltpu.async_copy(src_hbm.at[pl.ds(ci * chunk, chunk)],
                             srcb.at[par], csem)

        def wait_chunk(par):
            pltpu.make_async_copy(dst_hbm.at[pl.ds(0, chunk)],
                                  dstb.at[par], csem).wait()
            pltpu.make_async_copy(src_hbm.at[pl.ds(0, chunk)],
                                  srcb.at[par], csem).wait()

        def start_gather(base, par):
            def cp(k, _):
                v = pend[pl.ds(base + k * _L, _L)]
                idxb[par, pl.ds(k * _L, _L)] = jnp.clip(v >> 9, 0, n - 1)
                return 0
            lax.fori_loop(0, sub // _L, cp, 0, unroll=sub // _L)
            pltpu.async_copy(h_hbm.at[idxb.at[par]], rows.at[par], gsem)

        def wait_gather(par):
            pltpu.make_async_copy(h_hbm.at[idxb.at[par]],
                                  rows.at[par], gsem).wait()

        def accum_edge(packed16, par, jj, lane):
            pv = _dyn_gather(packed16, zeros16 + lane)
            addr = (pv & 511) * d + lane_iota
            idxs = [addr + (k * _L) for k in range(nk)]
            curs = [plsc.load_gather(acc, [ix]) for ix in idxs]
            hvs = [rows[par, jj, pl.ds(k * _L, _L)] for k in range(nk)]
            for k in range(nk):
                plsc.store_scatter(acc, [idxs[k]],
                                   jnp.maximum(curs[k], hvs[k]))

        def accum_group(base, par, gq):
            packed16 = pend[pl.ds(base + gq * _L, _L)]
            for lane in range(_L):
                accum_edge(packed16, par, gq * _L + lane, lane)

        def accum(base, par, jj):
            q16 = (jj // _L) * _L
            lane = jj - q16
            packed16 = pend[pl.ds(base + q16, _L)]
            accum_edge(packed16, par, jj, lane)

        start_chunk(0, 0)

        def chunk_body(ci, cnt):
            par = lax.rem(ci, 2)
            wait_chunk(par)

            @pl.when(ci + 1 < nchunks)
            def _pref():
                start_chunk(ci + 1, lax.rem(ci + 1, 2))

            def scan_body(g8, cnt):
                ms, packs, cvecs = [], [], []
                for u in range(8):
                    g = g8 * 8 + u
                    d16 = dstb[par, pl.ds(g * _L, _L)]
                    s16 = srcb[par, pl.ds(g * _L, _L)]
                    t = d16 - lo
                    m = (d16 >= lo) & (t < npt)
                    ms.append(m)
                    packs.append((s16 << 9) + t)
                    pc = plsc.all_reduce_population_count(m)
                    cvecs.append(pc if u == 0 else cvecs[u - 1] + pc)
                for u in range(8):
                    off = cnt if u == 0 else cnt + cvecs[u - 1][0]
                    plsc.store_compressed(pend.at[pl.ds(off, _L)],
                                          packs[u], mask=ms[u])
                return cnt + cvecs[7][0]

            cnt = lax.fori_loop(0, ng // 8, scan_body, cnt)
            nsub = cnt // sub

            @pl.when(nsub > 0)
            def _drain():
                start_gather(0, 0)

                def sub_body(b, _):
                    gpar = lax.rem(b, 2)
                    wait_gather(gpar)

                    @pl.when(b + 1 < nsub)
                    def _p():
                        start_gather((b + 1) * sub, lax.rem(b + 1, 2))

                    def grp_body(gq, _):
                        accum_group(b * sub, gpar, gq)
                        return 0
                    lax.fori_loop(0, sub // _L, grp_body, 0)
                    return 0

                lax.fori_loop(0, nsub, sub_body, 0)

            done = nsub * sub
            rem = cnt - done

            @pl.when(done > 0)
            def _mv():
                def mv(k, _):
                    v1 = pend[pl.ds(done + k * _L, _L)]
                    pend[pl.ds(k * _L, _L)] = v1
                    return 0
                lax.fori_loop(0, sub // _L, mv, 0, unroll=sub // _L)
            return rem

        cnt = lax.fori_loop(0, nchunks, chunk_body, jnp.int32(0))

        @pl.when(cnt > 0)
        def _flush():
            start_gather(0, 0)
            wait_gather(0)

            def edge_body(jj, _):
                accum(0, 0, jj)
                return 0
            lax.fori_loop(0, cnt, edge_body, 0)

        pltpu.sync_copy(acc.at[pl.ds(0, npt * d)],
                        out_hbm.at[pl.ds(lo * d, npt * d)])

    return agg_kernel(h, src_p, dst_p)


def _seg(Mt, v):
    return lax.dot_general(Mt, v, (((1,), (0,)), ((), ())), precision=_HI)


def _bcast(Mt, v):
    return lax.dot_general(Mt, v, (((0,), (0,)), ((), ())), precision=_HI)


def _graph_norm(x, Mt, inv_cnt, w, b, ms):
    mean = _seg(Mt, x) * inv_cnt
    t = x - _bcast(Mt, mean) * ms
    var = _seg(Mt, t * t) * inv_cnt
    std = jnp.sqrt(var + _EPS)
    return w * t / _bcast(Mt, std) + b


def _one_hot_t(batch1n, g):
    gi = lax.broadcasted_iota(jnp.int32, (g, 1), 0)
    return (batch1n == gi).astype(jnp.float32)


def _inv_cnt(Mt, n):
    ones = jnp.ones((n, 1), jnp.float32)
    cnt = lax.dot_general(Mt, ones, (((1,), (0,)), ((), ())), precision=_HI)
    return 1.0 / jnp.maximum(cnt, 1.0)


def _k1_body(x_ref, batch_ref, w_ref, b_ref, ms_ref, Wr_ref, bl_ref,
             h_ref, r_ref):
    x = x_ref[...]
    Mt = _one_hot_t(batch_ref[...], NUM_GRAPHS)
    h = _graph_norm(x, Mt, _inv_cnt(Mt, x.shape[0]),
                    w_ref[...], b_ref[...], ms_ref[...])
    h_ref[...] = h
    r_ref[...] = x + jnp.dot(h, Wr_ref[...],
                             preferred_element_type=jnp.float32) + bl_ref[...]


def _k3_body(agg_ref, r_ref, Wl_ref, batch_ref, w_ref, b_ref, ms_ref,
             out_ref):
    agg = agg_ref[...]
    agg = jnp.where(jnp.isfinite(agg), agg, 0.0)
    pre = jnp.dot(agg, Wl_ref[...], preferred_element_type=jnp.float32) \
        + r_ref[...]
    h2 = jnp.maximum(pre, 0.0)
    Mt = _one_hot_t(batch_ref[...], NUM_GRAPHS)
    out_ref[...] = _graph_norm(h2, Mt, _inv_cnt(Mt, h2.shape[0]),
                               w_ref[...], b_ref[...], ms_ref[...])


def kernel(x, edge_index, batch, W_l, b_l, W_r,
           gn1_w, gn1_b, gn1_ms, gn2_w, gn2_b, gn2_ms):
    n, d = x.shape
    e = edge_index.shape[1]
    batch1n = batch.reshape(1, n)
    row = lambda v: v.reshape(1, d)

    h, r = pl.pallas_call(
        _k1_body,
        out_shape=(jax.ShapeDtypeStruct((n, d), jnp.float32),
                   jax.ShapeDtypeStruct((n, d), jnp.float32)),
    )(x, batch1n, row(gn1_w), row(gn1_b), row(gn1_ms), W_r, row(b_l))

    npt = (n + _NW - 1) // _NW
    npt = ((npt + 7) // 8) * 8
    chunk = 8192
    e_pad = ((e + chunk - 1) // chunk) * chunk
    src_p = jnp.concatenate(
        [edge_index[0], jnp.zeros((e_pad - e,), jnp.int32)])
    dst_p = jnp.concatenate(
        [edge_index[1], jnp.full((e_pad - e,), 1 << 20, jnp.int32)])
    agg_flat = _sc_agg(h, src_p, dst_p, npt, chunk, 64)
    agg = agg_flat.reshape(_NW * npt, d)[:n]

    out = pl.pallas_call(
        _k3_body,
        out_shape=jax.ShapeDtypeStruct((n, d), jnp.float32),
    )(agg, r, W_l, batch1n, row(gn2_w), row(gn2_b), row(gn2_ms))
    return out
